# Initial kernel scaffold; baseline (speedup 1.0000x reference)
import jax
import jax.numpy as jnp
from jax.experimental import pallas as pl


def kernel(x, l0_wx, l0_wh, l0_b_gates, l0_w_out, l0_b_out, l1_wx, l1_wh, l1_b_gates, l1_w_out, l1_b_out):
    raise NotImplementedError("write your pallas kernel here")


# speedup vs baseline: 15.6595x; 15.6595x over previous
import jax
import jax.numpy as jnp
from jax.experimental import pallas as pl
from jax.experimental.pallas import tpu as pltpu


def _shift_lanes(v, o):
    if o == 0:
        return v
    C, M = v.shape
    z = jnp.zeros((C, abs(o)), v.dtype)
    if o > 0:
        return jnp.concatenate([z, v[:, :M - o]], axis=1)
    return jnp.concatenate([v[:, -o:], z], axis=1)


def _make_scan_body(H, W, Ch, T, nb, prebuilt_px):
    M = H * W

    def body(px_ref, c0_ref, w_ref, b_ref, h_ref, c_ref):
        col = jax.lax.broadcasted_iota(jnp.int32, (1, M), 1) % W
        colmask = {-1: col >= 1, 1: col <= W - 2}

        def taps9(v):
            out = []
            for dy in (-1, 0, 1):
                for dx in (-1, 0, 1):
                    u = _shift_lanes(v, -(dy * W + dx))
                    if dx != 0:
                        u = jnp.where(colmask[dx], u, jnp.zeros_like(u))
                    out.append(u)
            return out

        carries = []
        for i in range(nb):
            carries.append((jnp.zeros((Ch, M), jnp.bfloat16), c0_ref[i]))
        for t in range(T):
            for i in range(nb):
                h_prev, c_prev = carries[i]
                ph = taps9(h_prev)
                if prebuilt_px:
                    patch = jnp.concatenate(ph + [px_ref[i, t]], axis=0)
                else:
                    patch = jnp.concatenate(ph + taps9(px_ref[i, t]), axis=0)
                g = jnp.dot(w_ref[...], patch,
                            preferred_element_type=jnp.float32)
                g = g + b_ref[...]
                i_g = jax.nn.sigmoid(g[0 * Ch:1 * Ch])
                f_g = jax.nn.sigmoid(g[1 * Ch:2 * Ch])
                o_g = jax.nn.sigmoid(g[2 * Ch:3 * Ch])
                g_g = jnp.tanh(g[3 * Ch:4 * Ch])
                c_new = f_g * c_prev + i_g * g_g
                h_new = (o_g * jnp.tanh(c_new)).astype(jnp.bfloat16)
                h_ref[i, t] = h_new
                carries[i] = (h_new, c_new)
        for i in range(nb):
            c_ref[i] = carries[i][1]

    return body


def _scan_call(px, c0, w, b, H, W, Ch, prebuilt, nb):
    B, T, Kdim, M = px.shape
    Kx = Kdim if prebuilt else 9 * Kdim
    flops = int(B * T * (2 * 4 * Ch * (Kx + 9 * Ch) * M + 10 * Ch * M))
    trans = int(5 * B * T * Ch * M)
    bytes_accessed = int(px.size * 2 + c0.size * 4 + w.size * 2 + b.size * 4
                         + B * T * Ch * M * 2 + B * Ch * M * 4)
    return pl.pallas_call(
        _make_scan_body(H, W, Ch, T, nb, prebuilt),
        out_shape=(jax.ShapeDtypeStruct((B, T, Ch, M), jnp.bfloat16),
                   jax.ShapeDtypeStruct((B, Ch, M), jnp.float32)),
        grid=(B // nb,),
        in_specs=[
            pl.BlockSpec((nb, T, Kdim, M), lambda g: (g, 0, 0, 0)),
            pl.BlockSpec((nb, Ch, M), lambda g: (g, 0, 0)),
            pl.BlockSpec(w.shape, lambda g: (0, 0)),
            pl.BlockSpec(b.shape, lambda g: (0, 0)),
        ],
        out_specs=(
            pl.BlockSpec((nb, T, Ch, M), lambda g: (g, 0, 0, 0)),
            pl.BlockSpec((nb, Ch, M), lambda g: (g, 0, 0)),
        ),
        compiler_params=pltpu.CompilerParams(
            dimension_semantics=("parallel",)),
        cost_estimate=pl.CostEstimate(flops=flops, transcendentals=trans,
                                      bytes_accessed=bytes_accessed),
    )(px, c0, w, b)


def _make_down_body(Ho, Wo, Ch, Cout, T1, nb):
    Mo = Ho * Wo

    def body(p_ref, w_ref, b_ref, o_ref):
        col = jax.lax.broadcasted_iota(jnp.int32, (1, Mo), 1) % Wo
        cmask = col >= 1
        for i in range(nb):
            for t in range(T1):
                taps = []
                for dy in (-1, 0, 1):
                    for dx in (-1, 0, 1):
                        pidx = (0 if dy == 0 else 2) + (0 if dx == 0 else 1)
                        o = (Wo if dy == -1 else 0) + (1 if dx == -1 else 0)
                        u = _shift_lanes(p_ref[i, t, pidx], o)
                        if dx == -1:
                            u = jnp.where(cmask, u, jnp.zeros_like(u))
                        taps.append(u)
                patch = jnp.concatenate(taps, axis=0)
                y = jnp.dot(w_ref[...], patch,
                            preferred_element_type=jnp.float32) + b_ref[...]
                o_ref[i, t] = jnp.where(y > 0.0, y, 0.2 * y)

    return body


def _down_call(planes, w, b, Ho, Wo, nb):
    B, T1, _, Ch, Mo = planes.shape
    Cout = w.shape[0]
    flops = int(2 * B * T1 * Cout * 9 * Ch * Mo)
    bytes_accessed = int(planes.size * 2 + w.size * 2 + b.size * 4
                         + B * T1 * Cout * Mo * 4)
    return pl.pallas_call(
        _make_down_body(Ho, Wo, Ch, Cout, T1, nb),
        out_shape=jax.ShapeDtypeStruct((B, T1, Cout, Mo), jnp.float32),
        grid=(B // nb,),
        in_specs=[
            pl.BlockSpec((nb, T1, 4, Ch, Mo), lambda g: (g, 0, 0, 0, 0)),
            pl.BlockSpec(w.shape, lambda g: (0, 0)),
            pl.BlockSpec(b.shape, lambda g: (0, 0)),
        ],
        out_specs=pl.BlockSpec((nb, T1, Cout, Mo), lambda g: (g, 0, 0, 0)),
        compiler_params=pltpu.CompilerParams(
            dimension_semantics=("parallel",)),
        cost_estimate=pl.CostEstimate(flops=flops, transcendentals=0,
                                      bytes_accessed=bytes_accessed),
    )(planes, w, b)


def _im2col_3x3_s1(x):
    B, T, C, H, W = x.shape
    xp = jnp.pad(x, ((0, 0), (0, 0), (0, 0), (1, 1), (1, 1)))
    cols = [xp[:, :, :, dy:dy + H, dx:dx + W]
            for dy in range(3) for dx in range(3)]
    return jnp.stack(cols, axis=2).reshape(B, T, 9 * C, H * W)


def _parity(y):
    B, T1, C, H, W = y.shape
    Ho, Wo = H // 2, W // 2
    y = y.reshape(B, T1, C, Ho, 2, Wo, 2)
    y = jnp.transpose(y, (0, 1, 4, 6, 2, 3, 5))
    return y.reshape(B, T1, 4, C, Ho * Wo)


def _layer(x_seq, raw_x, wx, wh, b_gates, w_out, b_out, H, W, c_init, nb,
           keep_c):
    B, T = x_seq.shape[:2]
    Ch = wh.shape[0] // 4
    w_cat = jnp.concatenate([wh, wx], axis=1).astype(jnp.bfloat16)
    h_all, c_last = _scan_call(x_seq, c_init, w_cat, b_gates, H, W, Ch,
                               not raw_x, nb)
    if keep_c:
        y = jnp.concatenate([h_all, c_last.astype(jnp.bfloat16)[:, None]],
                            axis=1)
    else:
        y = h_all
    planes = _parity(y.reshape(y.shape[0], y.shape[1], Ch, H, W))
    Ho, Wo = H // 2, W // 2
    out = _down_call(planes, w_out.astype(jnp.bfloat16),
                     b_out.reshape(-1, 1), Ho, Wo, nb)
    if keep_c:
        return out[:, :T], out[:, T]
    return out, None


def kernel(x, l0_wx, l0_wh, l0_b_gates, l0_w_out, l0_b_out,
           l1_wx, l1_wh, l1_b_gates, l1_w_out, l1_b_out):
    B, T, C0, H0, W0 = x.shape
    Ch0 = l0_wh.shape[0] // 4
    nb = 2

    px0 = _im2col_3x3_s1(x).astype(jnp.bfloat16)
    c0_init = jnp.zeros((B, Ch0, H0 * W0), jnp.float32)
    x1, c1 = _layer(px0, False, l0_wx, l0_wh, l0_b_gates, l0_w_out, l0_b_out,
                    H0, W0, c0_init, nb, keep_c=True)

    H1, W1 = H0 // 2, W0 // 2
    x1 = x1.astype(jnp.bfloat16)
    out, _ = _layer(x1, True, l1_wx, l1_wh, l1_b_gates, l1_w_out, l1_b_out,
                    H1, W1, c1, nb, keep_c=False)
    Cout = l1_w_out.shape[0]
    return out.reshape(B, T, Cout, H1 // 2, W1 // 2)


# speedup vs baseline: 15.9874x; 1.0209x over previous
import jax
import jax.numpy as jnp
from jax.experimental import pallas as pl
from jax.experimental.pallas import tpu as pltpu


def _shift_lanes(v, o):
    if o == 0:
        return v
    C, M = v.shape
    z = jnp.zeros((C, abs(o)), v.dtype)
    if o > 0:
        return jnp.concatenate([z, v[:, :M - o]], axis=1)
    return jnp.concatenate([v[:, -o:], z], axis=1)


def _make_scan_body(H, W, Ch, T, nb, prebuilt_px):
    M = H * W

    def body(px_ref, c0_ref, w_ref, b_ref, h_ref, c_ref):
        col = jax.lax.broadcasted_iota(jnp.int32, (1, M), 1) % W
        colmask = {-1: col >= 1, 1: col <= W - 2}

        def taps9(v):
            out = []
            for dy in (-1, 0, 1):
                for dx in (-1, 0, 1):
                    u = _shift_lanes(v, -(dy * W + dx))
                    if dx != 0:
                        u = jnp.where(colmask[dx], u, jnp.zeros_like(u))
                    out.append(u)
            return out

        carries = []
        for i in range(nb):
            carries.append((jnp.zeros((Ch, M), jnp.bfloat16), c0_ref[i]))
        for t in range(T):
            for i in range(nb):
                h_prev, c_prev = carries[i]
                ph = taps9(h_prev)
                if prebuilt_px:
                    patch = jnp.concatenate(ph + [px_ref[i, t]], axis=0)
                else:
                    patch = jnp.concatenate(ph + taps9(px_ref[i, t]), axis=0)
                g = jnp.dot(w_ref[...], patch,
                            preferred_element_type=jnp.float32)
                g = g + b_ref[...]
                i_g = jax.nn.sigmoid(g[0 * Ch:1 * Ch])
                f_g = jax.nn.sigmoid(g[1 * Ch:2 * Ch])
                o_g = jax.nn.sigmoid(g[2 * Ch:3 * Ch])
                g_g = jnp.tanh(g[3 * Ch:4 * Ch])
                c_new = f_g * c_prev + i_g * g_g
                h_new = (o_g * jnp.tanh(c_new)).astype(jnp.bfloat16)
                h_ref[i, t] = h_new
                carries[i] = (h_new, c_new)
        for i in range(nb):
            c_ref[i] = carries[i][1]

    return body


def _scan_call(px, c0, w, b, H, W, Ch, prebuilt, nb):
    B, T, Kdim, M = px.shape
    Kx = Kdim if prebuilt else 9 * Kdim
    flops = int(B * T * (2 * 4 * Ch * (Kx + 9 * Ch) * M + 10 * Ch * M))
    trans = int(5 * B * T * Ch * M)
    bytes_accessed = int(px.size * 2 + c0.size * 4 + w.size * 2 + b.size * 4
                         + B * T * Ch * M * 2 + B * Ch * M * 4)
    return pl.pallas_call(
        _make_scan_body(H, W, Ch, T, nb, prebuilt),
        out_shape=(jax.ShapeDtypeStruct((B, T, Ch, M), jnp.bfloat16),
                   jax.ShapeDtypeStruct((B, Ch, M), jnp.float32)),
        grid=(B // nb,),
        in_specs=[
            pl.BlockSpec((nb, T, Kdim, M), lambda g: (g, 0, 0, 0)),
            pl.BlockSpec((nb, Ch, M), lambda g: (g, 0, 0)),
            pl.BlockSpec(w.shape, lambda g: (0, 0)),
            pl.BlockSpec(b.shape, lambda g: (0, 0)),
        ],
        out_specs=(
            pl.BlockSpec((nb, T, Ch, M), lambda g: (g, 0, 0, 0)),
            pl.BlockSpec((nb, Ch, M), lambda g: (g, 0, 0)),
        ),
        compiler_params=pltpu.CompilerParams(
            dimension_semantics=("parallel",)),
        cost_estimate=pl.CostEstimate(flops=flops, transcendentals=trans,
                                      bytes_accessed=bytes_accessed),
    )(px, c0, w, b)


def _make_down_body(Ho, Wo, Ch, Cout, T1, nb):
    Mo = Ho * Wo

    def body(p_ref, w_ref, b_ref, o_ref):
        col = jax.lax.broadcasted_iota(jnp.int32, (1, Mo), 1) % Wo
        cmask = col >= 1
        for i in range(nb):
            for t in range(T1):
                taps = []
                for dy in (-1, 0, 1):
                    for dx in (-1, 0, 1):
                        pidx = (0 if dy == 0 else 2) + (0 if dx == 0 else 1)
                        o = (Wo if dy == -1 else 0) + (1 if dx == -1 else 0)
                        u = _shift_lanes(p_ref[i, t, pidx], o)
                        if dx == -1:
                            u = jnp.where(cmask, u, jnp.zeros_like(u))
                        taps.append(u)
                patch = jnp.concatenate(taps, axis=0)
                y = jnp.dot(w_ref[...], patch,
                            preferred_element_type=jnp.float32) + b_ref[...]
                o_ref[i, t] = jnp.where(y > 0.0, y, 0.2 * y)

    return body


def _down_call(planes, w, b, Ho, Wo, nb):
    B, T1, _, Ch, Mo = planes.shape
    Cout = w.shape[0]
    flops = int(2 * B * T1 * Cout * 9 * Ch * Mo)
    bytes_accessed = int(planes.size * 2 + w.size * 2 + b.size * 4
                         + B * T1 * Cout * Mo * 4)
    return pl.pallas_call(
        _make_down_body(Ho, Wo, Ch, Cout, T1, nb),
        out_shape=jax.ShapeDtypeStruct((B, T1, Cout, Mo), jnp.float32),
        grid=(B // nb,),
        in_specs=[
            pl.BlockSpec((nb, T1, 4, Ch, Mo), lambda g: (g, 0, 0, 0, 0)),
            pl.BlockSpec(w.shape, lambda g: (0, 0)),
            pl.BlockSpec(b.shape, lambda g: (0, 0)),
        ],
        out_specs=pl.BlockSpec((nb, T1, Cout, Mo), lambda g: (g, 0, 0, 0)),
        compiler_params=pltpu.CompilerParams(
            dimension_semantics=("parallel",)),
        cost_estimate=pl.CostEstimate(flops=flops, transcendentals=0,
                                      bytes_accessed=bytes_accessed),
    )(planes, w, b)


def _im2col_3x3_s1(x):
    B, T, C, H, W = x.shape
    xp = jnp.pad(x, ((0, 0), (0, 0), (0, 0), (1, 1), (1, 1)))
    cols = [xp[:, :, :, dy:dy + H, dx:dx + W]
            for dy in range(3) for dx in range(3)]
    return jnp.stack(cols, axis=2).reshape(B, T, 9 * C, H * W)


def _parity(y):
    B, T1, C, H, W = y.shape
    Ho, Wo = H // 2, W // 2
    y = y.reshape(B, T1, C, Ho, 2, Wo, 2)
    y = jnp.transpose(y, (0, 1, 4, 6, 2, 3, 5))
    return y.reshape(B, T1, 4, C, Ho * Wo)


def _layer(x_seq, raw_x, wx, wh, b_gates, w_out, b_out, H, W, c_init,
           nb_scan, nb, keep_c):
    B, T = x_seq.shape[:2]
    Ch = wh.shape[0] // 4
    w_cat = jnp.concatenate([wh, wx], axis=1).astype(jnp.bfloat16)
    h_all, c_last = _scan_call(x_seq, c_init, w_cat, b_gates, H, W, Ch,
                               not raw_x, nb_scan)
    if keep_c:
        y = jnp.concatenate([h_all, c_last.astype(jnp.bfloat16)[:, None]],
                            axis=1)
    else:
        y = h_all
    planes = _parity(y.reshape(y.shape[0], y.shape[1], Ch, H, W))
    Ho, Wo = H // 2, W // 2
    out = _down_call(planes, w_out.astype(jnp.bfloat16),
                     b_out.reshape(-1, 1), Ho, Wo, nb)
    if keep_c:
        return out[:, :T], out[:, T]
    return out, None


def kernel(x, l0_wx, l0_wh, l0_b_gates, l0_w_out, l0_b_out,
           l1_wx, l1_wh, l1_b_gates, l1_w_out, l1_b_out):
    B, T, C0, H0, W0 = x.shape
    Ch0 = l0_wh.shape[0] // 4
    nb_scan, nb = 4, 2

    px0 = _im2col_3x3_s1(x).astype(jnp.bfloat16)
    c0_init = jnp.zeros((B, Ch0, H0 * W0), jnp.float32)
    x1, c1 = _layer(px0, False, l0_wx, l0_wh, l0_b_gates, l0_w_out, l0_b_out,
                    H0, W0, c0_init, nb_scan, nb, keep_c=True)

    H1, W1 = H0 // 2, W0 // 2
    x1 = x1.astype(jnp.bfloat16)
    out, _ = _layer(x1, True, l1_wx, l1_wh, l1_b_gates, l1_w_out, l1_b_out,
                    H1, W1, c1, nb_scan, nb, keep_c=False)
    Cout = l1_w_out.shape[0]
    return out.reshape(B, T, Cout, H1 // 2, W1 // 2)
